# baseline (speedup 1.0000x reference)
import functools

import jax
import jax.numpy as jnp
from jax import lax
from jax.experimental import pallas as pl
from jax.experimental.pallas import tpu as pltpu
from jax.experimental.pallas import tpu_sc as plsc

N_NODES = 10000
N_PAD = 10240
E_TOTAL = 320000
H = 128
NC, NS, L = 2, 16, 16
NW = NC * NS
EPW = E_TOTAL // NW
C = 80
NCHUNK = EPW // C
NEG_INF = float("-inf")

_SC_PARAMS = pltpu.CompilerParams(needs_layout_passes=False)
_GDN = lax.GatherDimensionNumbers(
    offset_dims=(), collapsed_slice_dims=(0,), start_index_map=(0,))


def _iota16():
    return lax.iota(jnp.int32, L)


def _take16(x, idx):
    return lax.gather(x, idx[:, None], _GDN, (1,),
                      mode=lax.GatherScatterMode.PROMISE_IN_BOUNDS)


def _bcast_lane(x, lane):
    return _take16(x, jnp.full((L,), lane, jnp.int32))


def _seg_update(tab_ref, k16, v16, op):
    ks, vs = plsc.sort_key_val(k16, v16)
    it = _iota16()
    prev = _take16(ks, jnp.maximum(it - 1, 0))
    is_start = (it == 0) | (ks != prev)
    segid = plsc.cumsum(is_start.astype(jnp.int32))
    v = vs
    for d in (1, 2, 4, 8):
        idx = jnp.maximum(it - d, 0)
        same = (it >= d) & (segid == _take16(segid, idx))
        vd = _take16(v, idx)
        v = jnp.where(same, op(v, vd), v)
    nxt = _take16(ks, jnp.minimum(it + 1, L - 1))
    is_end = (it == L - 1) | (ks != nxt)
    cur = plsc.load_gather(tab_ref, [ks])
    plsc.store_scatter(tab_ref, [ks], op(cur, v), mask=is_end)


def _fill1d(ref, n, value):
    def body(i, _):
        ref[pl.ds(i * L, L)] = jnp.full((L,), value, jnp.float32)
        return _
    lax.fori_loop(0, n // L, body, None, unroll=8)


def _merge_tables(part_h, glob_v, tmp_v, op):
    pltpu.sync_copy(part_h.at[0], glob_v)

    def tbl_body(t, _):
        pltpu.sync_copy(part_h.at[t], tmp_v)

        def red_body(i, _):
            sl = pl.ds(i * L, L)
            glob_v[sl] = op(glob_v[sl], tmp_v[sl])
            return _
        lax.fori_loop(0, N_PAD // L, red_body, None, unroll=8)
        return _
    lax.fori_loop(1, NW, tbl_body, None)


def _wid():
    return lax.axis_index("s") * NC + lax.axis_index("c")


def _tree_dot(sb, rb, db, e):
    parts = []
    for j in range(H // L):
        sl = pl.ds(j * L, L)
        parts.append((sb[e, sl] + rb[e, sl]) * db[e, sl])
    while len(parts) > 1:
        parts = [parts[i] + parts[i + 1] for i in range(0, len(parts), 2)]
    return jnp.sum(parts[0])


def _k1_body(ent, rel_emb, src_h, dst_h, rel_h, norm_out, mpart_out,
             src_v, dst_v, rel_v, norm_v, mloc_v,
             s0, r0, d0, s1, r1, d1, sem0, sem1):
    wid = _wid()
    ebase = wid * EPW

    pltpu.sync_copy(src_h.at[pl.ds(ebase, EPW)], src_v)
    pltpu.sync_copy(dst_h.at[pl.ds(ebase, EPW)], dst_v)
    pltpu.sync_copy(rel_h.at[pl.ds(ebase, EPW)], rel_v)
    _fill1d(mloc_v, N_PAD, NEG_INF)

    def issue(g, sb, rb, db, sem):
        cb = g * C
        pltpu.async_copy(ent.at[src_v.at[pl.ds(cb, C)]], sb, sem)
        pltpu.async_copy(rel_emb.at[rel_v.at[pl.ds(cb, C)]], rb, sem)
        pltpu.async_copy(ent.at[dst_v.at[pl.ds(cb, C)]], db, sem)

    def wait3(sb, rb, db, sem):
        pltpu.make_async_copy(ent.at[pl.ds(0, C)], sb, sem).wait()
        pltpu.make_async_copy(ent.at[pl.ds(0, C)], rb, sem).wait()
        pltpu.make_async_copy(ent.at[pl.ds(0, C)], db, sem).wait()

    def compute(sb, rb, db, g):
        cb = g * C

        def q_body(q, norms):
            e0 = q * 8
            lane0 = lax.rem(e0, L)
            it = _iota16()
            for c in range(8):
                norms = jnp.where(it == lane0 + c,
                                  _tree_dot(sb, rb, db, e0 + c), norms)

            @pl.when(lane0 == L - 8)
            def _flush():
                norm_v[pl.ds(cb + e0 - (L - 8), L)] = norms
            return norms

        lax.fori_loop(0, C // 8, q_body, jnp.zeros((L,), jnp.float32))

    issue(0, s0, r0, d0, sem0)

    def pair_body(u, _):
        g0 = 2 * u
        wait3(s0, r0, d0, sem0)
        issue(g0 + 1, s1, r1, d1, sem1)
        compute(s0, r0, d0, g0)
        wait3(s1, r1, d1, sem1)
        issue(g0 + 2, s0, r0, d0, sem0)
        compute(s1, r1, d1, g0 + 1)
        return _
    lax.fori_loop(0, (NCHUNK - 1) // 2, pair_body, None)
    wait3(s0, r0, d0, sem0)
    compute(s0, r0, d0, NCHUNK - 1)

    def seg_body(i, _):
        d16 = dst_v[pl.ds(i * L, L)]
        n16 = norm_v[pl.ds(i * L, L)]
        _seg_update(mloc_v, d16, n16, jnp.maximum)
        return _
    lax.fori_loop(0, EPW // L, seg_body, None, unroll=2)

    pltpu.sync_copy(norm_v, norm_out.at[pl.ds(ebase, EPW)])
    pltpu.sync_copy(mloc_v, mpart_out.at[wid])


def _k2_body(norm_h, dst_h, mglob_h, e_out, spart_out,
             dst_v, norm_v, mglob_v, sloc_v):
    wid = _wid()
    ebase = wid * EPW

    pltpu.sync_copy(dst_h.at[pl.ds(ebase, EPW)], dst_v)
    pltpu.sync_copy(norm_h.at[pl.ds(ebase, EPW)], norm_v)
    pltpu.sync_copy(mglob_h, mglob_v)
    _fill1d(sloc_v, N_PAD, 0.0)

    def grp_body(i, _):
        sl = pl.ds(i * L, L)
        d16 = dst_v[sl]
        m16 = plsc.load_gather(mglob_v, [d16])
        e16 = jnp.exp(norm_v[sl] - m16)
        norm_v[sl] = e16
        _seg_update(sloc_v, d16, e16, jnp.add)
        return _
    lax.fori_loop(0, EPW // L, grp_body, None)

    pltpu.sync_copy(norm_v, e_out.at[pl.ds(ebase, EPW)])
    pltpu.sync_copy(sloc_v, spart_out.at[wid])


C3 = 40
SCE = 400
NSUP = EPW // SCE
NIN = SCE // C3


def _k3_body(ent, rel_emb, src_h, dst_h, rel_h, e_h, sglob_h, npart_out,
             srcc_v, relc_v, dstc_v, evc_v, sglob_v,
             s0, r0, i0, s1, r1, i1, msg0, msg1, neigh_sh,
             sem0, sem1, scs0, scs1):
    cid = lax.axis_index("c")
    sid = lax.axis_index("s")
    ebase = (sid * NC + cid) * EPW

    pltpu.sync_copy(sglob_h, sglob_v)
    msg = msg0

    def zrow_body(r, _):
        for j in range(H // L):
            msg[r, pl.ds(j * L, L)] = jnp.zeros((L,), jnp.float32)
        return _
    lax.fori_loop(0, C3, zrow_body, None)
    for k in range(N_PAD // NS // C3):
        pltpu.sync_copy(msg, neigh_sh.at[pl.ds(sid * (N_PAD // NS) + k * C3, C3)])
    plsc.subcore_barrier()

    def sup_body(u, _):
        sb = ebase + u * SCE
        pltpu.sync_copy(src_h.at[pl.ds(sb, SCE)], srcc_v)
        pltpu.sync_copy(rel_h.at[pl.ds(sb, SCE)], relc_v)
        pltpu.sync_copy(dst_h.at[pl.ds(sb, SCE)], dstc_v)
        pltpu.sync_copy(e_h.at[pl.ds(sb, SCE)], evc_v)

        def alpha_body(i, _):
            sl = pl.ds(i * L, L)
            s16 = plsc.load_gather(sglob_v, [dstc_v[sl]])
            evc_v[sl] = evc_v[sl] / (s16 + 1e-16)
            return _
        lax.fori_loop(0, SCE // L, alpha_body, None, unroll=2)

        def issue(g, sbuf, rbuf, ibuf, sem):
            cb = g * C3
            pltpu.async_copy(ent.at[srcc_v.at[pl.ds(cb, C3)]], sbuf, sem)
            pltpu.async_copy(rel_emb.at[relc_v.at[pl.ds(cb, C3)]], rbuf, sem)
            pltpu.async_copy(dst_h.at[pl.ds(sb + cb, C3)], ibuf, sem)

        def wait3(sbuf, rbuf, ibuf, sem):
            pltpu.make_async_copy(ent.at[pl.ds(0, C3)], sbuf, sem).wait()
            pltpu.make_async_copy(ent.at[pl.ds(0, C3)], rbuf, sem).wait()
            pltpu.make_async_copy(dst_h.at[pl.ds(0, C3)], ibuf, sem).wait()

        def compute(sbuf, rbuf, ibuf, g, msgb, scs, first):
            cb = g * C3

            @pl.when(jnp.logical_not(first))
            def _drain():
                pltpu.make_async_copy(npart_out.at[0, pl.ds(0, C3)], msgb,
                                      scs).wait()

            def q_body(q, _q):
                e0 = q * 4
                for c in range(4):
                    a16 = plsc.load_gather(
                        evc_v, [jnp.full((L,), cb + e0 + c, jnp.int32)])
                    for j in range(H // L):
                        sl = pl.ds(j * L, L)
                        msgb[e0 + c, sl] = (sbuf[e0 + c, sl]
                                            + rbuf[e0 + c, sl]) * a16
                return _q
            lax.fori_loop(0, C3 // 4, q_body, None)
            pltpu.async_copy(msgb, neigh_sh.at[ibuf], scs, add=True)

        issue(0, s0, r0, i0, sem0)

        def pair_body(p, _p):
            g0 = 2 * p
            wait3(s0, r0, i0, sem0)
            issue(g0 + 1, s1, r1, i1, sem1)
            compute(s0, r0, i0, g0, msg0, scs0, p == 0)
            wait3(s1, r1, i1, sem1)

            @pl.when(g0 + 2 < NIN)
            def _nxt():
                issue(g0 + 2, s0, r0, i0, sem0)
            compute(s1, r1, i1, g0 + 1, msg1, scs1, p == 0)
            return _p
        lax.fori_loop(0, NIN // 2, pair_body, None)
        pltpu.make_async_copy(npart_out.at[0, pl.ds(0, C3)], msg0, scs0).wait()
        pltpu.make_async_copy(npart_out.at[0, pl.ds(0, C3)], msg1, scs1).wait()
        return _
    lax.fori_loop(0, NSUP, sup_body, None)

    plsc.subcore_barrier()
    rows = pl.ds(sid * (N_PAD // NS), N_PAD // NS)
    pltpu.sync_copy(neigh_sh.at[rows], npart_out.at[cid, rows])


def _merge_max_body(p_ref, o_ref):
    o_ref[...] = jnp.max(p_ref[...], axis=0)


def _merge_sum_body(p_ref, o_ref):
    o_ref[...] = jnp.sum(p_ref[...], axis=0)


def _tc_merge(part, body):
    return pl.pallas_call(
        body,
        out_shape=jax.ShapeDtypeStruct((N_PAD,), jnp.float32),
    )(part)


def _sc_k12_impl(ent, rel_emb, src, dst, rel):
    mesh = plsc.VectorSubcoreMesh(core_axis_name="c", subcore_axis_name="s")
    norm, mpart = pl.kernel(
        _k1_body,
        out_type=[
            jax.ShapeDtypeStruct((E_TOTAL,), jnp.float32),
            jax.ShapeDtypeStruct((NW, N_PAD), jnp.float32),
        ],
        mesh=mesh,
        compiler_params=_SC_PARAMS,
        scratch_types=[
            pltpu.VMEM((EPW,), jnp.int32),
            pltpu.VMEM((EPW,), jnp.int32),
            pltpu.VMEM((EPW,), jnp.int32),
            pltpu.VMEM((EPW,), jnp.float32),
            pltpu.VMEM((N_PAD,), jnp.float32),
            pltpu.VMEM((C, H), jnp.float32),
            pltpu.VMEM((C, H), jnp.float32),
            pltpu.VMEM((C, H), jnp.float32),
            pltpu.VMEM((C, H), jnp.float32),
            pltpu.VMEM((C, H), jnp.float32),
            pltpu.VMEM((C, H), jnp.float32),
            pltpu.SemaphoreType.DMA,
            pltpu.SemaphoreType.DMA,
        ],
    )(ent, rel_emb, src, dst, rel)

    mglob = _tc_merge(mpart, _merge_max_body)
    ev, spart = pl.kernel(
        _k2_body,
        out_type=[
            jax.ShapeDtypeStruct((E_TOTAL,), jnp.float32),
            jax.ShapeDtypeStruct((NW, N_PAD), jnp.float32),
        ],
        mesh=mesh,
        compiler_params=_SC_PARAMS,
        scratch_types=[
            pltpu.VMEM((EPW,), jnp.int32),
            pltpu.VMEM((EPW,), jnp.float32),
            pltpu.VMEM((N_PAD,), jnp.float32),
            pltpu.VMEM((N_PAD,), jnp.float32),
        ],
    )(norm, dst, mglob)
    return ev, spart


@jax.jit
def _sc_k12(ent, rel_emb, src, dst, rel):
    return _sc_k12_impl(ent, rel_emb, src, dst, rel)


@jax.jit
def _sc_pipeline(ent, rel_emb, src, dst, rel):
    mesh = plsc.VectorSubcoreMesh(core_axis_name="c", subcore_axis_name="s")
    ev, spart = _sc_k12_impl(ent, rel_emb, src, dst, rel)
    sglob = _tc_merge(spart, _merge_sum_body)
    npart = pl.kernel(
        _k3_body,
        out_type=jax.ShapeDtypeStruct((NC, N_PAD, H), jnp.float32),
        mesh=mesh,
        compiler_params=_SC_PARAMS,
        scratch_types=[
            pltpu.VMEM((SCE,), jnp.int32),
            pltpu.VMEM((SCE,), jnp.int32),
            pltpu.VMEM((SCE,), jnp.int32),
            pltpu.VMEM((SCE,), jnp.float32),
            pltpu.VMEM((N_PAD,), jnp.float32),
            pltpu.VMEM((C3, H), jnp.float32),
            pltpu.VMEM((C3, H), jnp.float32),
            pltpu.VMEM((C3,), jnp.int32),
            pltpu.VMEM((C3, H), jnp.float32),
            pltpu.VMEM((C3, H), jnp.float32),
            pltpu.VMEM((C3,), jnp.int32),
            pltpu.VMEM((C3, H), jnp.float32),
            pltpu.VMEM((C3, H), jnp.float32),
            pltpu.VMEM_SHARED((N_PAD, H), jnp.float32),
            pltpu.SemaphoreType.DMA,
            pltpu.SemaphoreType.DMA,
            pltpu.SemaphoreType.DMA,
            pltpu.SemaphoreType.DMA,
        ],
    )(ent, rel_emb, src, dst, rel, ev, sglob)
    return npart


_BLK = 512


def _k4_body(p_ref, w_ref, o_ref):
    x = p_ref[0] + p_ref[1]
    o_ref[...] = jnp.tanh(
        jnp.dot(x, w_ref[...], preferred_element_type=jnp.float32))


@jax.jit
def _project(npart, neigh_w):
    return pl.pallas_call(
        _k4_body,
        grid=(N_PAD // _BLK,),
        in_specs=[
            pl.BlockSpec((NC, _BLK, H), lambda i: (0, i, 0)),
            pl.BlockSpec((H, H), lambda i: (0, 0)),
        ],
        out_specs=pl.BlockSpec((_BLK, H), lambda i: (i, 0)),
        out_shape=jax.ShapeDtypeStruct((N_PAD, H), jnp.float32),
    )(npart, neigh_w)


def kernel(ent_emb, rel_emb, neigh_w, edge_index, rel_id):
    src = edge_index[0]
    dst = edge_index[1]
    npart = _sc_pipeline(ent_emb, rel_emb, src, dst, rel_id)
    out = _project(npart, neigh_w)
    return out[:N_NODES]


# speedup vs baseline: 1.0410x; 1.0410x over previous
import functools

import jax
import jax.numpy as jnp
from jax import lax
from jax.experimental import pallas as pl
from jax.experimental.pallas import tpu as pltpu
from jax.experimental.pallas import tpu_sc as plsc

N_NODES = 10000
N_PAD = 10240
E_TOTAL = 320000
H = 128
NC, NS, L = 2, 16, 16
NW = NC * NS
EPW = E_TOTAL // NW
C = 80
NCHUNK = EPW // C
NEG_INF = float("-inf")

_SC_PARAMS = pltpu.CompilerParams(needs_layout_passes=False)
_GDN = lax.GatherDimensionNumbers(
    offset_dims=(), collapsed_slice_dims=(0,), start_index_map=(0,))


def _iota16():
    return lax.iota(jnp.int32, L)


def _take16(x, idx):
    return lax.gather(x, idx[:, None], _GDN, (1,),
                      mode=lax.GatherScatterMode.PROMISE_IN_BOUNDS)


def _seg_update(tab_ref, k16, v16, op):
    ks, vs = plsc.sort_key_val(k16, v16)
    it = _iota16()
    prev = _take16(ks, jnp.maximum(it - 1, 0))
    is_start = (it == 0) | (ks != prev)
    segid = plsc.cumsum(is_start.astype(jnp.int32))
    v = vs
    for d in (1, 2, 4, 8):
        idx = jnp.maximum(it - d, 0)
        same = (it >= d) & (segid == _take16(segid, idx))
        vd = _take16(v, idx)
        v = jnp.where(same, op(v, vd), v)
    nxt = _take16(ks, jnp.minimum(it + 1, L - 1))
    is_end = (it == L - 1) | (ks != nxt)
    cur = plsc.load_gather(tab_ref, [ks])
    plsc.store_scatter(tab_ref, [ks], op(cur, v), mask=is_end)


def _fill1d(ref, n, value):
    def body(i, _):
        ref[pl.ds(i * L, L)] = jnp.full((L,), value, jnp.float32)
        return _
    lax.fori_loop(0, n // L, body, None, unroll=8)


def _wid():
    return lax.axis_index("s") * NC + lax.axis_index("c")


def _tree_dot(sb, rb, db, e):
    parts = []
    for j in range(H // L):
        sl = pl.ds(j * L, L)
        parts.append((sb[e, sl] + rb[e, sl]) * db[e, sl])
    while len(parts) > 1:
        parts = [parts[i] + parts[i + 1] for i in range(0, len(parts), 2)]
    return jnp.sum(parts[0])


def _k1_body(ent, rel_emb, src_h, dst_h, rel_h, norm_out, mpart_out,
             src_v, dst_v, rel_v, norm_v, mloc_v,
             s0, r0, d0, s1, r1, d1, sem0, sem1):
    wid = _wid()
    ebase = wid * EPW

    pltpu.sync_copy(src_h.at[pl.ds(ebase, EPW)], src_v)
    pltpu.sync_copy(dst_h.at[pl.ds(ebase, EPW)], dst_v)
    pltpu.sync_copy(rel_h.at[pl.ds(ebase, EPW)], rel_v)
    _fill1d(mloc_v, N_PAD, NEG_INF)

    def issue(g, sb, rb, db, sem):
        cb = g * C
        pltpu.async_copy(ent.at[src_v.at[pl.ds(cb, C)]], sb, sem)
        pltpu.async_copy(rel_emb.at[rel_v.at[pl.ds(cb, C)]], rb, sem)
        pltpu.async_copy(ent.at[dst_v.at[pl.ds(cb, C)]], db, sem)

    def wait3(sb, rb, db, sem):
        pltpu.make_async_copy(ent.at[pl.ds(0, C)], sb, sem).wait()
        pltpu.make_async_copy(ent.at[pl.ds(0, C)], rb, sem).wait()
        pltpu.make_async_copy(ent.at[pl.ds(0, C)], db, sem).wait()

    def compute(sb, rb, db, g):
        cb = g * C

        def q_body(q, norms):
            e0 = q * 4
            lane0 = lax.rem(e0, L)
            it = _iota16()
            for c in range(4):
                norms = jnp.where(it == lane0 + c,
                                  _tree_dot(sb, rb, db, e0 + c), norms)

            @pl.when(lane0 == L - 4)
            def _flush():
                norm_v[pl.ds(cb + e0 - (L - 4), L)] = norms
            return norms

        lax.fori_loop(0, C // 4, q_body, jnp.zeros((L,), jnp.float32))

    issue(0, s0, r0, d0, sem0)

    def pair_body(u, _):
        g0 = 2 * u
        wait3(s0, r0, d0, sem0)
        issue(g0 + 1, s1, r1, d1, sem1)
        compute(s0, r0, d0, g0)
        wait3(s1, r1, d1, sem1)
        issue(g0 + 2, s0, r0, d0, sem0)
        compute(s1, r1, d1, g0 + 1)
        return _
    lax.fori_loop(0, (NCHUNK - 1) // 2, pair_body, None)
    wait3(s0, r0, d0, sem0)
    compute(s0, r0, d0, NCHUNK - 1)

    def seg_body(i, _):
        d16 = dst_v[pl.ds(i * L, L)]
        n16 = norm_v[pl.ds(i * L, L)]
        _seg_update(mloc_v, d16, n16, jnp.maximum)
        return _
    lax.fori_loop(0, EPW // L, seg_body, None)

    pltpu.sync_copy(norm_v, norm_out.at[pl.ds(ebase, EPW)])
    pltpu.sync_copy(mloc_v, mpart_out.at[wid])


def _k2_body(norm_h, dst_h, mglob_h, e_out, spart_out,
             dst_v, norm_v, mglob_v, sloc_v):
    wid = _wid()
    ebase = wid * EPW

    pltpu.sync_copy(dst_h.at[pl.ds(ebase, EPW)], dst_v)
    pltpu.sync_copy(norm_h.at[pl.ds(ebase, EPW)], norm_v)
    pltpu.sync_copy(mglob_h, mglob_v)
    _fill1d(sloc_v, N_PAD, 0.0)

    def grp_body(i, _):
        sl = pl.ds(i * L, L)
        d16 = dst_v[sl]
        m16 = plsc.load_gather(mglob_v, [d16])
        e16 = jnp.exp(norm_v[sl] - m16)
        norm_v[sl] = e16
        _seg_update(sloc_v, d16, e16, jnp.add)
        return _
    lax.fori_loop(0, EPW // L, grp_body, None)

    pltpu.sync_copy(norm_v, e_out.at[pl.ds(ebase, EPW)])
    pltpu.sync_copy(sloc_v, spart_out.at[wid])


C3 = 40
SCE = 400
NSUP = EPW // SCE
NIN = SCE // C3


def _k3_body(ent, rel_emb, src_h, dst_h, rel_h, e_h, sglob_h, npart_out,
             srcc_v, relc_v, dstc_v, evc_v, sglob_v,
             s0, r0, i0, s1, r1, i1, msg0, msg1, neigh_sh,
             sem0, sem1, scs0, scs1):
    cid = lax.axis_index("c")
    sid = lax.axis_index("s")
    ebase = (sid * NC + cid) * EPW

    pltpu.sync_copy(sglob_h, sglob_v)
    msg = msg0

    def zrow_body(r, _):
        for j in range(H // L):
            msg[r, pl.ds(j * L, L)] = jnp.zeros((L,), jnp.float32)
        return _
    lax.fori_loop(0, C3, zrow_body, None)
    for k in range(N_PAD // NS // C3):
        pltpu.sync_copy(msg, neigh_sh.at[pl.ds(sid * (N_PAD // NS) + k * C3, C3)])
    plsc.subcore_barrier()

    def sup_body(u, _):
        sb = ebase + u * SCE
        pltpu.sync_copy(src_h.at[pl.ds(sb, SCE)], srcc_v)
        pltpu.sync_copy(rel_h.at[pl.ds(sb, SCE)], relc_v)
        pltpu.sync_copy(dst_h.at[pl.ds(sb, SCE)], dstc_v)
        pltpu.sync_copy(e_h.at[pl.ds(sb, SCE)], evc_v)

        def alpha_body(i, _):
            sl = pl.ds(i * L, L)
            s16 = plsc.load_gather(sglob_v, [dstc_v[sl]])
            evc_v[sl] = evc_v[sl] / (s16 + 1e-16)
            return _
        lax.fori_loop(0, SCE // L, alpha_body, None, unroll=2)

        def issue(g, sbuf, rbuf, ibuf, sem):
            cb = g * C3
            pltpu.async_copy(ent.at[srcc_v.at[pl.ds(cb, C3)]], sbuf, sem)
            pltpu.async_copy(rel_emb.at[relc_v.at[pl.ds(cb, C3)]], rbuf, sem)
            pltpu.async_copy(dst_h.at[pl.ds(sb + cb, C3)], ibuf, sem)

        def wait3(sbuf, rbuf, ibuf, sem):
            pltpu.make_async_copy(ent.at[pl.ds(0, C3)], sbuf, sem).wait()
            pltpu.make_async_copy(ent.at[pl.ds(0, C3)], rbuf, sem).wait()
            pltpu.make_async_copy(dst_h.at[pl.ds(0, C3)], ibuf, sem).wait()

        def compute(sbuf, rbuf, ibuf, g, msgb, scs, first):
            cb = g * C3

            @pl.when(jnp.logical_not(first))
            def _drain():
                pltpu.make_async_copy(npart_out.at[0, pl.ds(0, C3)], msgb,
                                      scs).wait()

            def q_body(q, _q):
                e0 = q * 4
                for c in range(4):
                    a16 = plsc.load_gather(
                        evc_v, [jnp.full((L,), cb + e0 + c, jnp.int32)])
                    for j in range(H // L):
                        sl = pl.ds(j * L, L)
                        msgb[e0 + c, sl] = (sbuf[e0 + c, sl]
                                            + rbuf[e0 + c, sl]) * a16
                return _q
            lax.fori_loop(0, C3 // 4, q_body, None)
            pltpu.async_copy(msgb, neigh_sh.at[ibuf], scs, add=True)

        issue(0, s0, r0, i0, sem0)

        def pair_body(p, _p):
            g0 = 2 * p
            wait3(s0, r0, i0, sem0)
            issue(g0 + 1, s1, r1, i1, sem1)
            compute(s0, r0, i0, g0, msg0, scs0, p == 0)
            wait3(s1, r1, i1, sem1)

            @pl.when(g0 + 2 < NIN)
            def _nxt():
                issue(g0 + 2, s0, r0, i0, sem0)
            compute(s1, r1, i1, g0 + 1, msg1, scs1, p == 0)
            return _p
        lax.fori_loop(0, NIN // 2, pair_body, None)
        pltpu.make_async_copy(npart_out.at[0, pl.ds(0, C3)], msg0, scs0).wait()
        pltpu.make_async_copy(npart_out.at[0, pl.ds(0, C3)], msg1, scs1).wait()
        return _
    lax.fori_loop(0, NSUP, sup_body, None)

    plsc.subcore_barrier()
    rows = pl.ds(sid * (N_PAD // NS), N_PAD // NS)
    pltpu.sync_copy(neigh_sh.at[rows], npart_out.at[cid, rows])


def _merge_max_body(p_ref, o_ref):
    o_ref[...] = jnp.max(p_ref[...], axis=0)


def _merge_sum_body(p_ref, o_ref):
    o_ref[...] = jnp.sum(p_ref[...], axis=0)


def _tc_merge(part, body):
    return pl.pallas_call(
        body,
        out_shape=jax.ShapeDtypeStruct((N_PAD,), jnp.float32),
    )(part)


def _sc_k12_impl(ent, rel_emb, src, dst, rel):
    mesh = plsc.VectorSubcoreMesh(core_axis_name="c", subcore_axis_name="s")
    norm, mpart = pl.kernel(
        _k1_body,
        out_type=[
            jax.ShapeDtypeStruct((E_TOTAL,), jnp.float32),
            jax.ShapeDtypeStruct((NW, N_PAD), jnp.float32),
        ],
        mesh=mesh,
        compiler_params=_SC_PARAMS,
        scratch_types=[
            pltpu.VMEM((EPW,), jnp.int32),
            pltpu.VMEM((EPW,), jnp.int32),
            pltpu.VMEM((EPW,), jnp.int32),
            pltpu.VMEM((EPW,), jnp.float32),
            pltpu.VMEM((N_PAD,), jnp.float32),
            pltpu.VMEM((C, H), jnp.float32),
            pltpu.VMEM((C, H), jnp.float32),
            pltpu.VMEM((C, H), jnp.float32),
            pltpu.VMEM((C, H), jnp.float32),
            pltpu.VMEM((C, H), jnp.float32),
            pltpu.VMEM((C, H), jnp.float32),
            pltpu.SemaphoreType.DMA,
            pltpu.SemaphoreType.DMA,
        ],
    )(ent, rel_emb, src, dst, rel)

    mglob = _tc_merge(mpart, _merge_max_body)
    ev, spart = pl.kernel(
        _k2_body,
        out_type=[
            jax.ShapeDtypeStruct((E_TOTAL,), jnp.float32),
            jax.ShapeDtypeStruct((NW, N_PAD), jnp.float32),
        ],
        mesh=mesh,
        compiler_params=_SC_PARAMS,
        scratch_types=[
            pltpu.VMEM((EPW,), jnp.int32),
            pltpu.VMEM((EPW,), jnp.float32),
            pltpu.VMEM((N_PAD,), jnp.float32),
            pltpu.VMEM((N_PAD,), jnp.float32),
        ],
    )(norm, dst, mglob)
    return ev, spart


@jax.jit
def _sc_pipeline(ent, rel_emb, src, dst, rel):
    mesh = plsc.VectorSubcoreMesh(core_axis_name="c", subcore_axis_name="s")
    ev, spart = _sc_k12_impl(ent, rel_emb, src, dst, rel)
    sglob = _tc_merge(spart, _merge_sum_body)
    npart = pl.kernel(
        _k3_body,
        out_type=jax.ShapeDtypeStruct((NC, N_PAD, H), jnp.float32),
        mesh=mesh,
        compiler_params=_SC_PARAMS,
        scratch_types=[
            pltpu.VMEM((SCE,), jnp.int32),
            pltpu.VMEM((SCE,), jnp.int32),
            pltpu.VMEM((SCE,), jnp.int32),
            pltpu.VMEM((SCE,), jnp.float32),
            pltpu.VMEM((N_PAD,), jnp.float32),
            pltpu.VMEM((C3, H), jnp.float32),
            pltpu.VMEM((C3, H), jnp.float32),
            pltpu.VMEM((C3,), jnp.int32),
            pltpu.VMEM((C3, H), jnp.float32),
            pltpu.VMEM((C3, H), jnp.float32),
            pltpu.VMEM((C3,), jnp.int32),
            pltpu.VMEM((C3, H), jnp.float32),
            pltpu.VMEM((C3, H), jnp.float32),
            pltpu.VMEM_SHARED((N_PAD, H), jnp.float32),
            pltpu.SemaphoreType.DMA,
            pltpu.SemaphoreType.DMA,
            pltpu.SemaphoreType.DMA,
            pltpu.SemaphoreType.DMA,
        ],
    )(ent, rel_emb, src, dst, rel, ev, sglob)
    return npart


_BLK = 512


def _k4_body(p_ref, w_ref, o_ref):
    x = p_ref[0] + p_ref[1]
    o_ref[...] = jnp.tanh(
        jnp.dot(x, w_ref[...], preferred_element_type=jnp.float32))


@jax.jit
def _project(npart, neigh_w):
    return pl.pallas_call(
        _k4_body,
        grid=(N_PAD // _BLK,),
        in_specs=[
            pl.BlockSpec((NC, _BLK, H), lambda i: (0, i, 0)),
            pl.BlockSpec((H, H), lambda i: (0, 0)),
        ],
        out_specs=pl.BlockSpec((_BLK, H), lambda i: (i, 0)),
        out_shape=jax.ShapeDtypeStruct((N_PAD, H), jnp.float32),
    )(npart, neigh_w)


def kernel(ent_emb, rel_emb, neigh_w, edge_index, rel_id):
    src = edge_index[0]
    dst = edge_index[1]
    npart = _sc_pipeline(ent_emb, rel_emb, src, dst, rel_id)
    out = _project(npart, neigh_w)
    return out[:N_NODES]


# speedup vs baseline: 1.0413x; 1.0003x over previous
import jax
import jax.numpy as jnp
from jax import lax
from jax.experimental import pallas as pl
from jax.experimental.pallas import tpu as pltpu
from jax.experimental.pallas import tpu_sc as plsc

N_NODES = 10000
N_PAD = 10240
E_TOTAL = 320000
H = 128
NC, NS, L = 2, 16, 16
NW = NC * NS
EPW = E_TOTAL // NW
C = 80
NCHUNK = EPW // C
NEG_INF = float("-inf")

_SC_PARAMS = pltpu.CompilerParams(needs_layout_passes=False)
_GDN = lax.GatherDimensionNumbers(
    offset_dims=(), collapsed_slice_dims=(0,), start_index_map=(0,))


def _iota16():
    return lax.iota(jnp.int32, L)


def _take16(x, idx):
    return lax.gather(x, idx[:, None], _GDN, (1,),
                      mode=lax.GatherScatterMode.PROMISE_IN_BOUNDS)


def _seg_update(tab_ref, k16, v16, op):
    ks, vs = plsc.sort_key_val(k16, v16)
    it = _iota16()
    prev = _take16(ks, jnp.maximum(it - 1, 0))
    is_start = (it == 0) | (ks != prev)
    segid = plsc.cumsum(is_start.astype(jnp.int32))
    v = vs
    for d in (1, 2, 4, 8):
        idx = jnp.maximum(it - d, 0)
        same = (it >= d) & (segid == _take16(segid, idx))
        vd = _take16(v, idx)
        v = jnp.where(same, op(v, vd), v)
    nxt = _take16(ks, jnp.minimum(it + 1, L - 1))
    is_end = (it == L - 1) | (ks != nxt)
    cur = plsc.load_gather(tab_ref, [ks])
    plsc.store_scatter(tab_ref, [ks], op(cur, v), mask=is_end)


def _fill1d(ref, n, value):
    def body(i, _):
        ref[pl.ds(i * L, L)] = jnp.full((L,), value, jnp.float32)
        return _
    lax.fori_loop(0, n // L, body, None, unroll=8)


def _wid():
    return lax.axis_index("s") * NC + lax.axis_index("c")


def _tree_dot(sb, rb, db, e):
    parts = []
    for j in range(H // L):
        sl = pl.ds(j * L, L)
        parts.append((sb[e, sl] + rb[e, sl]) * db[e, sl])
    while len(parts) > 1:
        parts = [parts[i] + parts[i + 1] for i in range(0, len(parts), 2)]
    return jnp.sum(parts[0])


def _k1_body(ent, rel_emb, src_h, dst_h, rel_h, norm_out, mpart_out,
             src_v, dst_v, rel_v, norm_v, mloc_v,
             s0, r0, d0, s1, r1, d1, sem0, sem1):
    wid = _wid()
    ebase = wid * EPW

    pltpu.sync_copy(src_h.at[pl.ds(ebase, EPW)], src_v)
    pltpu.sync_copy(dst_h.at[pl.ds(ebase, EPW)], dst_v)
    pltpu.sync_copy(rel_h.at[pl.ds(ebase, EPW)], rel_v)
    _fill1d(mloc_v, N_PAD, NEG_INF)

    def issue(g, sb, rb, db, sem):
        cb = g * C
        pltpu.async_copy(ent.at[src_v.at[pl.ds(cb, C)]], sb, sem)
        pltpu.async_copy(rel_emb.at[rel_v.at[pl.ds(cb, C)]], rb, sem)
        pltpu.async_copy(ent.at[dst_v.at[pl.ds(cb, C)]], db, sem)

    def wait3(sb, rb, db, sem):
        pltpu.make_async_copy(ent.at[pl.ds(0, C)], sb, sem).wait()
        pltpu.make_async_copy(ent.at[pl.ds(0, C)], rb, sem).wait()
        pltpu.make_async_copy(ent.at[pl.ds(0, C)], db, sem).wait()

    def compute(sb, rb, db, g):
        cb = g * C

        def q_body(q, norms):
            e0 = q * 4
            lane0 = lax.rem(e0, L)
            it = _iota16()
            for c in range(4):
                norms = jnp.where(it == lane0 + c,
                                  _tree_dot(sb, rb, db, e0 + c), norms)

            @pl.when(lane0 == L - 4)
            def _flush():
                norm_v[pl.ds(cb + e0 - (L - 4), L)] = norms
            return norms

        lax.fori_loop(0, C // 4, q_body, jnp.zeros((L,), jnp.float32))

    issue(0, s0, r0, d0, sem0)

    def pair_body(u, _):
        g0 = 2 * u
        wait3(s0, r0, d0, sem0)
        issue(g0 + 1, s1, r1, d1, sem1)
        compute(s0, r0, d0, g0)
        wait3(s1, r1, d1, sem1)
        issue(g0 + 2, s0, r0, d0, sem0)
        compute(s1, r1, d1, g0 + 1)
        return _
    lax.fori_loop(0, (NCHUNK - 1) // 2, pair_body, None)
    wait3(s0, r0, d0, sem0)
    compute(s0, r0, d0, NCHUNK - 1)

    def seg_body(i, _):
        d16 = dst_v[pl.ds(i * L, L)]
        n16 = norm_v[pl.ds(i * L, L)]
        _seg_update(mloc_v, d16, n16, jnp.maximum)
        return _
    lax.fori_loop(0, EPW // L, seg_body, None)

    pltpu.sync_copy(norm_v, norm_out.at[pl.ds(ebase, EPW)])
    pltpu.sync_copy(mloc_v, mpart_out.at[wid])


def _k2_body(norm_h, dst_h, mglob_h, e_out, spart_out,
             dst_v, norm_v, mglob_v, sloc_v):
    wid = _wid()
    ebase = wid * EPW

    pltpu.sync_copy(dst_h.at[pl.ds(ebase, EPW)], dst_v)
    pltpu.sync_copy(norm_h.at[pl.ds(ebase, EPW)], norm_v)
    pltpu.sync_copy(mglob_h, mglob_v)
    _fill1d(sloc_v, N_PAD, 0.0)

    def grp_body(i, _):
        sl = pl.ds(i * L, L)
        d16 = dst_v[sl]
        m16 = plsc.load_gather(mglob_v, [d16])
        e16 = jnp.exp(norm_v[sl] - m16)
        norm_v[sl] = e16
        _seg_update(sloc_v, d16, e16, jnp.add)
        return _
    lax.fori_loop(0, EPW // L, grp_body, None)

    pltpu.sync_copy(norm_v, e_out.at[pl.ds(ebase, EPW)])
    pltpu.sync_copy(sloc_v, spart_out.at[wid])


C3 = 40
SCE = 400
NSUP = EPW // SCE
NIN = SCE // C3


def _k3_body(ent, rel_emb, src_h, dst_h, rel_h, e_h, sglob_h, npart_out,
             srcc_v, relc_v, dstc_v, evc_v, sglob_v,
             s0, r0, i0, s1, r1, i1, msg0, msg1, neigh_sh,
             sem0, sem1, scs0, scs1):
    cid = lax.axis_index("c")
    sid = lax.axis_index("s")
    ebase = (sid * NC + cid) * EPW

    pltpu.sync_copy(sglob_h, sglob_v)
    msg = msg0

    def zrow_body(r, _):
        for j in range(H // L):
            msg[r, pl.ds(j * L, L)] = jnp.zeros((L,), jnp.float32)
        return _
    lax.fori_loop(0, C3, zrow_body, None)
    for k in range(N_PAD // NS // C3):
        pltpu.sync_copy(msg, neigh_sh.at[pl.ds(sid * (N_PAD // NS) + k * C3, C3)])
    plsc.subcore_barrier()

    def sup_body(u, _):
        sb = ebase + u * SCE
        pltpu.sync_copy(src_h.at[pl.ds(sb, SCE)], srcc_v)
        pltpu.sync_copy(rel_h.at[pl.ds(sb, SCE)], relc_v)
        pltpu.sync_copy(dst_h.at[pl.ds(sb, SCE)], dstc_v)
        pltpu.sync_copy(e_h.at[pl.ds(sb, SCE)], evc_v)

        def alpha_body(i, _):
            sl = pl.ds(i * L, L)
            s16 = plsc.load_gather(sglob_v, [dstc_v[sl]])
            evc_v[sl] = evc_v[sl] / (s16 + 1e-16)
            return _
        lax.fori_loop(0, SCE // L, alpha_body, None, unroll=2)

        def issue(g, sbuf, rbuf, ibuf, sem):
            cb = g * C3
            pltpu.async_copy(ent.at[srcc_v.at[pl.ds(cb, C3)]], sbuf, sem)
            pltpu.async_copy(rel_emb.at[relc_v.at[pl.ds(cb, C3)]], rbuf, sem)
            pltpu.async_copy(dst_h.at[pl.ds(sb + cb, C3)], ibuf, sem)

        def wait3(sbuf, rbuf, ibuf, sem):
            pltpu.make_async_copy(ent.at[pl.ds(0, C3)], sbuf, sem).wait()
            pltpu.make_async_copy(ent.at[pl.ds(0, C3)], rbuf, sem).wait()
            pltpu.make_async_copy(dst_h.at[pl.ds(0, C3)], ibuf, sem).wait()

        def compute(sbuf, rbuf, ibuf, g, msgb, scs, first):
            cb = g * C3

            @pl.when(jnp.logical_not(first))
            def _drain():
                pltpu.make_async_copy(npart_out.at[0, pl.ds(0, C3)], msgb,
                                      scs).wait()

            def q_body(q, _q):
                e0 = q * 4
                for c in range(4):
                    a16 = plsc.load_gather(
                        evc_v, [jnp.full((L,), cb + e0 + c, jnp.int32)])
                    for j in range(H // L):
                        sl = pl.ds(j * L, L)
                        msgb[e0 + c, sl] = (sbuf[e0 + c, sl]
                                            + rbuf[e0 + c, sl]) * a16
                return _q
            lax.fori_loop(0, C3 // 4, q_body, None)
            pltpu.async_copy(msgb, neigh_sh.at[ibuf], scs, add=True)

        issue(0, s0, r0, i0, sem0)

        def pair_body(p, _p):
            g0 = 2 * p
            wait3(s0, r0, i0, sem0)
            issue(g0 + 1, s1, r1, i1, sem1)
            compute(s0, r0, i0, g0, msg0, scs0, p == 0)
            wait3(s1, r1, i1, sem1)

            @pl.when(g0 + 2 < NIN)
            def _nxt():
                issue(g0 + 2, s0, r0, i0, sem0)
            compute(s1, r1, i1, g0 + 1, msg1, scs1, p == 0)
            return _p
        lax.fori_loop(0, NIN // 2, pair_body, None)
        pltpu.make_async_copy(npart_out.at[0, pl.ds(0, C3)], msg0, scs0).wait()
        pltpu.make_async_copy(npart_out.at[0, pl.ds(0, C3)], msg1, scs1).wait()
        return _
    lax.fori_loop(0, NSUP, sup_body, None)

    plsc.subcore_barrier()
    rows = pl.ds(sid * (N_PAD // NS), N_PAD // NS)
    pltpu.sync_copy(neigh_sh.at[rows], npart_out.at[cid, rows])


def _merge_max_body(p_ref, o_ref):
    o_ref[...] = jnp.max(p_ref[...], axis=0)


def _merge_sum_body(p_ref, o_ref):
    o_ref[...] = jnp.sum(p_ref[...], axis=0)


def _tc_merge(part, body):
    return pl.pallas_call(
        body,
        out_shape=jax.ShapeDtypeStruct((N_PAD,), jnp.float32),
    )(part)


def _sc_k12_impl(ent, rel_emb, src, dst, rel):
    mesh = plsc.VectorSubcoreMesh(core_axis_name="c", subcore_axis_name="s")
    norm, mpart = pl.kernel(
        _k1_body,
        out_type=[
            jax.ShapeDtypeStruct((E_TOTAL,), jnp.float32),
            jax.ShapeDtypeStruct((NW, N_PAD), jnp.float32),
        ],
        mesh=mesh,
        compiler_params=_SC_PARAMS,
        scratch_types=[
            pltpu.VMEM((EPW,), jnp.int32),
            pltpu.VMEM((EPW,), jnp.int32),
            pltpu.VMEM((EPW,), jnp.int32),
            pltpu.VMEM((EPW,), jnp.float32),
            pltpu.VMEM((N_PAD,), jnp.float32),
            pltpu.VMEM((C, H), jnp.float32),
            pltpu.VMEM((C, H), jnp.float32),
            pltpu.VMEM((C, H), jnp.float32),
            pltpu.VMEM((C, H), jnp.float32),
            pltpu.VMEM((C, H), jnp.float32),
            pltpu.VMEM((C, H), jnp.float32),
            pltpu.SemaphoreType.DMA,
            pltpu.SemaphoreType.DMA,
        ],
    )(ent, rel_emb, src, dst, rel)

    mglob = _tc_merge(mpart, _merge_max_body)
    ev, spart = pl.kernel(
        _k2_body,
        out_type=[
            jax.ShapeDtypeStruct((E_TOTAL,), jnp.float32),
            jax.ShapeDtypeStruct((NW, N_PAD), jnp.float32),
        ],
        mesh=mesh,
        compiler_params=_SC_PARAMS,
        scratch_types=[
            pltpu.VMEM((EPW,), jnp.int32),
            pltpu.VMEM((EPW,), jnp.float32),
            pltpu.VMEM((N_PAD,), jnp.float32),
            pltpu.VMEM((N_PAD,), jnp.float32),
        ],
    )(norm, dst, mglob)
    return ev, spart


@jax.jit
def _sc_pipeline(ent, rel_emb, src, dst, rel):
    mesh = plsc.VectorSubcoreMesh(core_axis_name="c", subcore_axis_name="s")
    ev, spart = _sc_k12_impl(ent, rel_emb, src, dst, rel)
    sglob = _tc_merge(spart, _merge_sum_body)
    npart = pl.kernel(
        _k3_body,
        out_type=jax.ShapeDtypeStruct((NC, N_PAD, H), jnp.float32),
        mesh=mesh,
        compiler_params=_SC_PARAMS,
        scratch_types=[
            pltpu.VMEM((SCE,), jnp.int32),
            pltpu.VMEM((SCE,), jnp.int32),
            pltpu.VMEM((SCE,), jnp.int32),
            pltpu.VMEM((SCE,), jnp.float32),
            pltpu.VMEM((N_PAD,), jnp.float32),
            pltpu.VMEM((C3, H), jnp.float32),
            pltpu.VMEM((C3, H), jnp.float32),
            pltpu.VMEM((C3,), jnp.int32),
            pltpu.VMEM((C3, H), jnp.float32),
            pltpu.VMEM((C3, H), jnp.float32),
            pltpu.VMEM((C3,), jnp.int32),
            pltpu.VMEM((C3, H), jnp.float32),
            pltpu.VMEM((C3, H), jnp.float32),
            pltpu.VMEM_SHARED((N_PAD, H), jnp.float32),
            pltpu.SemaphoreType.DMA,
            pltpu.SemaphoreType.DMA,
            pltpu.SemaphoreType.DMA,
            pltpu.SemaphoreType.DMA,
        ],
    )(ent, rel_emb, src, dst, rel, ev, sglob)
    return npart


_BLK = 512


def _k4_body(p_ref, w_ref, o_ref):
    x = p_ref[0] + p_ref[1]
    o_ref[...] = jnp.tanh(
        jnp.dot(x, w_ref[...], preferred_element_type=jnp.float32))


@jax.jit
def _project(npart, neigh_w):
    return pl.pallas_call(
        _k4_body,
        grid=(N_PAD // _BLK,),
        in_specs=[
            pl.BlockSpec((NC, _BLK, H), lambda i: (0, i, 0)),
            pl.BlockSpec((H, H), lambda i: (0, 0)),
        ],
        out_specs=pl.BlockSpec((_BLK, H), lambda i: (i, 0)),
        out_shape=jax.ShapeDtypeStruct((N_PAD, H), jnp.float32),
    )(npart, neigh_w)


def kernel(ent_emb, rel_emb, neigh_w, edge_index, rel_id):
    src = edge_index[0]
    dst = edge_index[1]
    npart = _sc_pipeline(ent_emb, rel_emb, src, dst, rel_id)
    out = _project(npart, neigh_w)
    return out[:N_NODES]
